# Initial kernel scaffold; baseline (speedup 1.0000x reference)
import jax
import jax.numpy as jnp
from jax import lax
from jax.experimental import pallas as pl
from jax.experimental.pallas import tpu as pltpu
from jax.experimental.pallas import tpu_sc as plsc

N = 10000
E = 320000
F = 128
NPAD = 10240
NC = 2
NS = 16
NW = NC * NS
CHUNK = 128
CHUNKS = 80
GROUP = 8
EPT = CHUNKS * CHUNK
ROWS_PER_TILE = NPAD // NS


def _sc_degree(dstp, zeros1):
    mesh = plsc.VectorSubcoreMesh(core_axis_name="c", subcore_axis_name="s")

    def body(dst_hbm, zero_hbm, out_hbm, didx, ones_v, acc, sem):
        c = lax.axis_index("c")
        s = lax.axis_index("s")
        wid = c * NS + s
        pltpu.sync_copy(dst_hbm.at[wid], didx)
        for j in range(CHUNK // 16):
            ones_v[pl.ds(j * 16, 16)] = jnp.ones((16,), jnp.float32)

        @pl.when(s == 0)
        def _():
            pltpu.sync_copy(zero_hbm, acc)

        plsc.subcore_barrier()
        prev = []
        for g in range(CHUNKS // GROUP):
            for cp in prev:
                cp.wait()
            prev = [
                pltpu.async_copy(ones_v, acc.at[didx.at[g * GROUP + b]], sem,
                                 add=True)
                for b in range(GROUP)
            ]
        for cp in prev:
            cp.wait()
        plsc.subcore_barrier()
        sl = pl.ds(s * ROWS_PER_TILE, ROWS_PER_TILE)
        pltpu.sync_copy(acc.at[sl], out_hbm.at[c, sl])

    return pl.kernel(
        body,
        out_type=jax.ShapeDtypeStruct((NC, NPAD), jnp.float32),
        mesh=mesh,
        scratch_types=[
            pltpu.VMEM((CHUNKS, CHUNK), jnp.int32),
            pltpu.VMEM((CHUNK,), jnp.float32),
            pltpu.VMEM_SHARED((NPAD,), jnp.float32),
            pltpu.SemaphoreType.DMA,
        ],
    )(dstp, zeros1)


def _sc_edge_scatter(y, srcp, dstp, zerosf, feat):
    mesh = plsc.VectorSubcoreMesh(core_axis_name="c", subcore_axis_name="s")

    def body(y_hbm, src_hbm, dst_hbm, zero_hbm, out_hbm,
             sidx, didx, msgs, acc, sem_g, sem_s):
        c = lax.axis_index("c")
        s = lax.axis_index("s")
        wid = c * NS + s
        pltpu.sync_copy(src_hbm.at[wid], sidx)
        pltpu.sync_copy(dst_hbm.at[wid], didx)

        @pl.when(s == 0)
        def _():
            pltpu.sync_copy(zero_hbm, acc)

        plsc.subcore_barrier()
        prev = []
        for g in range(CHUNKS // GROUP):
            gcps = [
                pltpu.async_copy(y_hbm.at[sidx.at[g * GROUP + b]],
                                 msgs.at[g * GROUP + b], sem_g)
                for b in range(GROUP)
            ]
            for cp in gcps:
                cp.wait()
            for cp in prev:
                cp.wait()
            prev = [
                pltpu.async_copy(msgs.at[g * GROUP + b],
                                 acc.at[didx.at[g * GROUP + b]], sem_s,
                                 add=True)
                for b in range(GROUP)
            ]
        for cp in prev:
            cp.wait()
        plsc.subcore_barrier()
        sl = pl.ds(s * ROWS_PER_TILE, ROWS_PER_TILE)
        pltpu.sync_copy(acc.at[sl], out_hbm.at[c, sl])

    return pl.kernel(
        body,
        out_type=jax.ShapeDtypeStruct((NC, NPAD, feat), jnp.float32),
        mesh=mesh,
        scratch_types=[
            pltpu.VMEM((CHUNKS, CHUNK), jnp.int32),
            pltpu.VMEM((CHUNKS, CHUNK), jnp.int32),
            pltpu.VMEM((CHUNKS, CHUNK, feat), jnp.float32),
            pltpu.VMEM_SHARED((NPAD, feat), jnp.float32),
            pltpu.SemaphoreType.DMA,
            pltpu.SemaphoreType.DMA,
        ],
    )(y, srcp, dstp, zerosf)


def _tc_first(x, W1, degp):
    def body(x_ref, w_ref, degp_ref, y_ref, dis_ref):
        deg = degp_ref[0] + degp_ref[1] + 1.0
        dis = lax.rsqrt(deg)[:N]
        xw = jnp.dot(x_ref[...], w_ref[...],
                     preferred_element_type=jnp.float32)
        y_ref[...] = xw * dis
        dis_ref[...] = dis

    return pl.pallas_call(
        body,
        out_shape=[
            jax.ShapeDtypeStruct((N, 4), jnp.float32),
            jax.ShapeDtypeStruct((N, 1), jnp.float32),
        ],
    )(x, W1, degp)


def _tc_mid(accp, y, dis, b, W, feat_out):
    def body(accp_ref, y_ref, dis_ref, b_ref, w_ref, ynext_ref):
        acc = accp_ref[0, :N, :] + accp_ref[1, :N, :] + y_ref[...]
        h = jnp.tanh(acc * dis_ref[...] + b_ref[...])
        ynext_ref[...] = jnp.dot(h, w_ref[...],
                                 preferred_element_type=jnp.float32) * dis_ref[...]

    return pl.pallas_call(
        body,
        out_shape=jax.ShapeDtypeStruct((N, feat_out), jnp.float32),
    )(accp, y, dis, b, W)


def _tc_last(accp, y, dis, b, Wc, bc):
    def body(accp_ref, y_ref, dis_ref, b_ref, wc_ref, bc_ref, out_ref, h_ref):
        acc = accp_ref[0, :N, :] + accp_ref[1, :N, :] + y_ref[...]
        h = jnp.tanh(acc * dis_ref[...] + b_ref[...])
        h_ref[...] = h
        out_ref[...] = jnp.dot(h, wc_ref[...],
                               preferred_element_type=jnp.float32) + bc_ref[...]

    return pl.pallas_call(
        body,
        out_shape=[
            jax.ShapeDtypeStruct((N, 4), jnp.float32),
            jax.ShapeDtypeStruct((N, 2), jnp.float32),
        ],
    )(accp, y, dis, b, Wc, bc)


def kernel(x, edge_index, W1, b1, W2, b2, W3, b3, Wc, bc):
    src = edge_index[0].astype(jnp.int32)
    dst = edge_index[1].astype(jnp.int32)

    pade = NW * EPT - E
    ar = jnp.arange(pade, dtype=jnp.int32)
    pad_src = (ar * 131) % N
    pad_dst = N + ar % (NPAD - N)
    srcp = jnp.concatenate([src, pad_src]).reshape(NW, CHUNKS, CHUNK)
    dstp = jnp.concatenate([dst, pad_dst]).reshape(NW, CHUNKS, CHUNK)

    zeros1 = jnp.zeros((NPAD,), jnp.float32)
    zeros4 = jnp.zeros((NPAD, 4), jnp.float32)
    zeros2 = jnp.zeros((NPAD, 2), jnp.float32)

    degp = _sc_degree(dstp, zeros1)
    y1, dis = _tc_first(x, W1, degp.reshape(NC, NPAD, 1))

    acc1 = _sc_edge_scatter(y1, srcp, dstp, zeros4, 4)
    y2 = _tc_mid(acc1, y1, dis, b1.reshape(1, 4), W2, 4)

    acc2 = _sc_edge_scatter(y2, srcp, dstp, zeros4, 4)
    y3 = _tc_mid(acc2, y2, dis, b2.reshape(1, 4), W3, 2)

    acc3 = _sc_edge_scatter(y3, srcp, dstp, zeros2, 2)
    out, h = _tc_last(acc3, y3, dis, b3.reshape(1, 2), Wc, bc.reshape(1, 4))
    return (out, h)


# speedup vs baseline: 50.1440x; 50.1440x over previous
import jax
import jax.numpy as jnp
from jax import lax
from jax.experimental import pallas as pl
from jax.experimental.pallas import tpu as pltpu
from jax.experimental.pallas import tpu_sc as plsc

N = 10000
E = 320000
F = 128
NPAD = 10240
NC = 2
NS = 16
NW = NC * NS
CHUNK = 128
CHUNKS = 80
GROUP = 8
EPT = CHUNKS * CHUNK
ROWS_PER_TILE = NPAD // NS


def _sc_degree(dstp, zeros1):
    mesh = plsc.VectorSubcoreMesh(core_axis_name="c", subcore_axis_name="s")

    def body(dst_hbm, zero_hbm, out_hbm, didx, ones_v, acc, sem):
        c = lax.axis_index("c")
        s = lax.axis_index("s")
        wid = c * NS + s
        pltpu.sync_copy(dst_hbm.at[wid], didx)
        for j in range(CHUNK // 16):
            ones_v[pl.ds(j * 16, 16)] = jnp.ones((16,), jnp.float32)

        @pl.when(s == 0)
        def _():
            pltpu.sync_copy(zero_hbm, acc)

        plsc.subcore_barrier()
        prev = []
        for g in range(CHUNKS // GROUP):
            for cp in prev:
                cp.wait()
            prev = [
                pltpu.async_copy(ones_v, acc.at[didx.at[g * GROUP + b]], sem,
                                 add=True)
                for b in range(GROUP)
            ]
        for cp in prev:
            cp.wait()
        plsc.subcore_barrier()
        sl = pl.ds(s * ROWS_PER_TILE, ROWS_PER_TILE)
        pltpu.sync_copy(acc.at[sl], out_hbm.at[c, sl])

    return pl.kernel(
        body,
        out_type=jax.ShapeDtypeStruct((NC, NPAD), jnp.float32),
        mesh=mesh,
        scratch_types=[
            pltpu.VMEM((CHUNKS, CHUNK), jnp.int32),
            pltpu.VMEM((CHUNK,), jnp.float32),
            pltpu.VMEM_SHARED((NPAD,), jnp.float32),
            pltpu.SemaphoreType.DMA,
        ],
    )(dstp, zeros1)


def _sc_edge_scatter(y, srcp, dstp, zerosf, feat):
    mesh = plsc.VectorSubcoreMesh(core_axis_name="c", subcore_axis_name="s")

    def body(y_hbm, src_hbm, dst_hbm, zero_hbm, out_hbm,
             sidx, didx, msgs, acc, sem_g, sem_s):
        c = lax.axis_index("c")
        s = lax.axis_index("s")
        wid = c * NS + s
        pltpu.sync_copy(src_hbm.at[wid], sidx)
        pltpu.sync_copy(dst_hbm.at[wid], didx)

        @pl.when(s == 0)
        def _():
            pltpu.sync_copy(zero_hbm, acc)

        plsc.subcore_barrier()
        prev = []
        for g in range(CHUNKS // GROUP):
            gcps = [
                pltpu.async_copy(y_hbm.at[sidx.at[g * GROUP + b]],
                                 msgs.at[g * GROUP + b], sem_g)
                for b in range(GROUP)
            ]
            for cp in gcps:
                cp.wait()
            for cp in prev:
                cp.wait()
            prev = [
                pltpu.async_copy(msgs.at[g * GROUP + b],
                                 acc.at[didx.at[g * GROUP + b]], sem_s,
                                 add=True)
                for b in range(GROUP)
            ]
        for cp in prev:
            cp.wait()
        plsc.subcore_barrier()
        sl = pl.ds(s * ROWS_PER_TILE, ROWS_PER_TILE)
        pltpu.sync_copy(acc.at[sl], out_hbm.at[c, sl])

    return pl.kernel(
        body,
        out_type=jax.ShapeDtypeStruct((NC, NPAD, feat), jnp.float32),
        mesh=mesh,
        compiler_params=pltpu.CompilerParams(use_tc_tiling_on_sc=False),
        scratch_types=[
            pltpu.VMEM((CHUNKS, CHUNK), jnp.int32),
            pltpu.VMEM((CHUNKS, CHUNK), jnp.int32),
            pltpu.VMEM((CHUNKS, CHUNK, feat), jnp.float32),
            pltpu.VMEM_SHARED((NPAD, feat), jnp.float32),
            pltpu.SemaphoreType.DMA,
            pltpu.SemaphoreType.DMA,
        ],
    )(y, srcp, dstp, zerosf)


def _tc_first(x, W1, degp):
    def body(x_ref, w_ref, degp_ref, y_ref, dis_ref):
        deg = degp_ref[0] + degp_ref[1] + 1.0
        dis = lax.rsqrt(deg)[:N]
        xw = jnp.dot(x_ref[...], w_ref[...],
                     preferred_element_type=jnp.float32)
        y_ref[...] = xw * dis
        dis_ref[...] = dis

    return pl.pallas_call(
        body,
        out_shape=[
            jax.ShapeDtypeStruct((N, 4), jnp.float32),
            jax.ShapeDtypeStruct((N, 1), jnp.float32),
        ],
    )(x, W1, degp)


def _tc_mid(accp, y, dis, b, W, feat_out):
    def body(accp_ref, y_ref, dis_ref, b_ref, w_ref, ynext_ref):
        acc = accp_ref[0, :N, :] + accp_ref[1, :N, :] + y_ref[...]
        h = jnp.tanh(acc * dis_ref[...] + b_ref[...])
        ynext_ref[...] = jnp.dot(h, w_ref[...],
                                 preferred_element_type=jnp.float32) * dis_ref[...]

    return pl.pallas_call(
        body,
        out_shape=jax.ShapeDtypeStruct((N, feat_out), jnp.float32),
    )(accp, y, dis, b, W)


def _tc_last(accp, y, dis, b, Wc, bc):
    def body(accp_ref, y_ref, dis_ref, b_ref, wc_ref, bc_ref, out_ref, h_ref):
        acc = accp_ref[0, :N, :] + accp_ref[1, :N, :] + y_ref[...]
        h = jnp.tanh(acc * dis_ref[...] + b_ref[...])
        h_ref[...] = h
        out_ref[...] = jnp.dot(h, wc_ref[...],
                               preferred_element_type=jnp.float32) + bc_ref[...]

    return pl.pallas_call(
        body,
        out_shape=[
            jax.ShapeDtypeStruct((N, 4), jnp.float32),
            jax.ShapeDtypeStruct((N, 2), jnp.float32),
        ],
    )(accp, y, dis, b, Wc, bc)


def kernel(x, edge_index, W1, b1, W2, b2, W3, b3, Wc, bc):
    src = edge_index[0].astype(jnp.int32)
    dst = edge_index[1].astype(jnp.int32)

    pade = NW * EPT - E
    ar = jnp.arange(pade, dtype=jnp.int32)
    pad_src = (ar * 131) % N
    pad_dst = N + ar % (NPAD - N)
    srcp = jnp.concatenate([src, pad_src]).reshape(NW, CHUNKS, CHUNK)
    dstp = jnp.concatenate([dst, pad_dst]).reshape(NW, CHUNKS, CHUNK)

    zeros1 = jnp.zeros((NPAD,), jnp.float32)
    zeros4 = jnp.zeros((NPAD, 4), jnp.float32)
    zeros2 = jnp.zeros((NPAD, 2), jnp.float32)

    degp = _sc_degree(dstp, zeros1)
    y1, dis = _tc_first(x, W1, degp.reshape(NC, NPAD, 1))

    acc1 = _sc_edge_scatter(y1, srcp, dstp, zeros4, 4)
    y2 = _tc_mid(acc1, y1, dis, b1.reshape(1, 4), W2, 4)

    acc2 = _sc_edge_scatter(y2, srcp, dstp, zeros4, 4)
    y3 = _tc_mid(acc2, y2, dis, b2.reshape(1, 4), W3, 2)

    acc3 = _sc_edge_scatter(y3, srcp, dstp, zeros2, 2)
    out, h = _tc_last(acc3, y3, dis, b3.reshape(1, 2), Wc, bc.reshape(1, 4))
    return (out, h)
